# baseline (speedup 1.0000x reference)
import functools

import jax
import jax.numpy as jnp
from jax import lax
from jax.experimental import pallas as pl
from jax.experimental.pallas import tpu as pltpu

H = 64


def _dense_kernel(x_ref, w_ref, b_ref, o_ref):
    o_ref[...] = (
        jnp.dot(x_ref[...], w_ref[...], preferred_element_type=jnp.float32)
        + b_ref[...]
    )


def _dense(x, w_t, b, bn):
    n, fi = x.shape
    fo = w_t.shape[1]
    return pl.pallas_call(
        _dense_kernel,
        grid=(n // bn,),
        in_specs=[
            pl.BlockSpec((bn, fi), lambda i: (i, 0)),
            pl.BlockSpec((fi, fo), lambda i: (0, 0)),
            pl.BlockSpec((1, fo), lambda i: (0, 0)),
        ],
        out_specs=pl.BlockSpec((bn, fo), lambda i: (i, 0)),
        out_shape=jax.ShapeDtypeStruct((n, fo), jnp.float32),
    )(x, w_t, b.reshape(1, fo))


def _edge_kernel(ngroups, ea_ref, xj_ref, w_ref, b_ref, o_ref, a_ref):
    gi = H // ngroups
    xj = xj_ref[...]
    acc = jnp.zeros(o_ref.shape, jnp.float32)
    for g in range(ngroups):
        a_ref[...] = jnp.maximum(
            jnp.dot(
                ea_ref[...],
                w_ref[:, g * gi * H:(g + 1) * gi * H],
                preferred_element_type=jnp.float32,
            )
            + b_ref[:, g * gi * H:(g + 1) * gi * H],
            0.0,
        )
        for k in range(gi):
            i = g * gi + k
            acc = acc + a_ref[:, k * H:(k + 1) * H] * xj[:, i:i + 1]
    o_ref[...] = acc


def _edge_msgs(ea, xj, fce_w_t, fce_b, be=640, ngroups=8):
    e = ea.shape[0]
    fe = ea.shape[1]
    gi = H // ngroups
    return pl.pallas_call(
        functools.partial(_edge_kernel, ngroups),
        grid=(e // be,),
        in_specs=[
            pl.BlockSpec((be, fe), lambda i: (i, 0)),
            pl.BlockSpec((be, H), lambda i: (i, 0)),
            pl.BlockSpec((fe, H * H), lambda i: (0, 0)),
            pl.BlockSpec((1, H * H), lambda i: (0, 0)),
        ],
        out_specs=pl.BlockSpec((be, H), lambda i: (i, 0)),
        out_shape=jax.ShapeDtypeStruct((e, H), jnp.float32),
        scratch_shapes=[pltpu.VMEM((be, gi * H), jnp.float32)],
    )(ea, xj, fce_w_t, fce_b.reshape(1, H * H))


def _update_kernel(agg_ref, h_ref, root_ref, bias_ref, g_ref, b_ref, o_ref):
    s = (
        agg_ref[0]
        + agg_ref[1]
        + jnp.dot(h_ref[...], root_ref[...], preferred_element_type=jnp.float32)
        + bias_ref[...]
    )
    mu = jnp.mean(s, axis=-1, keepdims=True)
    var = jnp.mean((s - mu) ** 2, axis=-1, keepdims=True)
    ln = (s - mu) * jax.lax.rsqrt(var + 1e-5) * g_ref[...] + b_ref[...]
    o_ref[...] = jnp.maximum(ln, 0.0)


def _update(agg2, h, root, bias, g, b, bn=1000):
    n = h.shape[0]
    return pl.pallas_call(
        _update_kernel,
        grid=(n // bn,),
        in_specs=[
            pl.BlockSpec((2, bn, H), lambda i: (0, i, 0)),
            pl.BlockSpec((bn, H), lambda i: (i, 0)),
            pl.BlockSpec((H, H), lambda i: (0, 0)),
            pl.BlockSpec((1, H), lambda i: (0, 0)),
            pl.BlockSpec((1, H), lambda i: (0, 0)),
            pl.BlockSpec((1, H), lambda i: (0, 0)),
        ],
        out_specs=pl.BlockSpec((bn, H), lambda i: (i, 0)),
        out_shape=jax.ShapeDtypeStruct((n, H), jnp.float32),
    )(agg2, h, root, bias.reshape(1, H), g.reshape(1, H), b.reshape(1, H))


def kernel(x, edge_index, edge_attr, fc1_w, fc1_b, fce1_w, fce1_b, root1, bias1,
           gn1_g, gn1_b, fce2_w, fce2_b, root2, bias2, gn2_g, gn2_b, fc2_w, fc2_b):
    n = x.shape[0]
    src = edge_index[0]
    dst = edge_index[1]

    hx = _dense(x, fc1_w.T, fc1_b, bn=1000)

    def nnconv_agg(h, fce_w, fce_b):
        xj = jnp.take(h, src, axis=0)
        msg = _edge_msgs(edge_attr, xj, fce_w.T, fce_b)
        agg = jax.ops.segment_sum(msg, dst, num_segments=n)
        return jnp.stack([agg, jnp.zeros_like(agg)])

    agg2 = nnconv_agg(hx, fce1_w, fce1_b)
    h1 = _update(agg2, hx, root1, bias1, gn1_g, gn1_b)
    agg2 = nnconv_agg(h1, fce2_w, fce2_b)
    h2 = _update(agg2, h1, root2, bias2, gn2_g, gn2_b)
    out = _dense(h2, fc2_w.T, fc2_b, bn=1000)
    return out
